# baseline (speedup 1.0000x reference)
import functools

import jax
import jax.numpy as jnp
import numpy as np
from jax import lax
from jax.experimental import pallas as pl
from jax.experimental.pallas import tpu as pltpu
from jax.experimental.pallas import tpu_sc as plsc

_N = 10000
_D = 128
_H = _D // 2
_E = 320000

_NC = 2
_NS = 16
_EPS = _E // _NS
_K = 128
_NBLK = 160
_NBUF = 8
_NBLKP = _NBLK + _NBUF
_EPAD = _NBLKP * _K - _EPS
_CHK = 624
_TAIL = _N - _NS * _CHK

_P64 = np.r_[np.arange(0, 16), np.arange(32, 48),
             np.arange(16, 32), np.arange(48, 64)]


def _bf16_round_bits(u):
    one = jnp.uint32(1)
    r = u + jnp.uint32(0x7FFF) + ((u >> 16) & one)
    return r & jnp.uint32(0xFFFF0000)


def _project_body(x_ref, wp_ref, bp_ref, wh_ref, ba_ref,
                  o_ref, lo_ref, hi_ref):
    acc = (
        jnp.dot(x_ref[...], wp_ref[...], preferred_element_type=jnp.float32)
        + bp_ref[...]
    )
    o_ref[...] = (
        jnp.dot(acc, wh_ref[...], preferred_element_type=jnp.float32)
        + ba_ref[...]
    )
    bits = _bf16_round_bits(lax.bitcast_convert_type(acc, jnp.uint32))
    lo = (bits[:, 0:32] >> 16) | (bits[:, 32:64])
    hi = (bits[:, 64:96] >> 16) | (bits[:, 96:128])
    lo_ref[...] = lax.bitcast_convert_type(lo, jnp.int32)
    hi_ref[...] = lax.bitcast_convert_type(hi, jnp.int32)


def _project(features, W_proj, b_proj2, W_h, b_agg2):
    blk = 1000
    return pl.pallas_call(
        _project_body,
        grid=(_N // blk,),
        in_specs=[
            pl.BlockSpec((blk, _D), lambda i: (i, 0)),
            pl.BlockSpec((_D, _D), lambda i: (0, 0)),
            pl.BlockSpec((1, _D), lambda i: (0, 0)),
            pl.BlockSpec((_D, _D), lambda i: (0, 0)),
            pl.BlockSpec((1, _D), lambda i: (0, 0)),
        ],
        out_specs=[
            pl.BlockSpec((blk, _D), lambda i: (i, 0)),
            pl.BlockSpec((blk, _H // 2), lambda i: (i, 0)),
            pl.BlockSpec((blk, _H // 2), lambda i: (i, 0)),
        ],
        out_shape=[
            jax.ShapeDtypeStruct((_N, _D), jnp.float32),
            jax.ShapeDtypeStruct((_N, _H // 2), jnp.int32),
            jax.ShapeDtypeStruct((_N, _H // 2), jnp.int32),
        ],
    )(features, W_proj, b_proj2, W_h, b_agg2)


_mesh = plsc.VectorSubcoreMesh(core_axis_name="c", subcore_axis_name="s")


@functools.partial(
    pl.kernel,
    out_type=tuple(
        jax.ShapeDtypeStruct((_N, _H), jnp.float32) for _ in range(2)
    ),
    mesh=_mesh,
    compiler_params=pltpu.CompilerParams(use_tc_tiling_on_sc=False),
    scratch_types=[
        pltpu.VMEM((2, _NBUF, _K), jnp.int32),
        pltpu.VMEM((2, _NBUF, _K), jnp.int32),
        pltpu.VMEM((2, _NBUF, _K), jnp.float32),
        pltpu.VMEM((_NBUF, _K, _H // 2), jnp.int32),
        pltpu.VMEM((2, _K, _H), jnp.float32),
        pltpu.VMEM_SHARED((_N, _H), jnp.float32),
    ] + [pltpu.SemaphoreType.DMA] * (_NBUF + 4),
)
def _aggregate(src_hbm, dst_hbm, ew_hbm, hp_hbm,
               out0, out1,
               src_v2, dst_v2, ew_v2, rows4, rows_f2, agg_sh,
               *sems):
    c = lax.axis_index("c")
    s = lax.axis_index("s")

    zeros = jnp.zeros((16,), jnp.float32)
    zstage = rows_f2.at[0]

    def zrow(r, carry):
        for d in range(_H // 16):
            zstage[r, pl.ds(d * 16, 16)] = zeros
        return carry

    lax.fori_loop(0, _K, zrow, 0)

    for t in range(_CHK // _K):
        pltpu.sync_copy(zstage, agg_sh.at[pl.ds(s * _CHK + t * _K, _K)])
    pltpu.sync_copy(zstage.at[pl.ds(0, _CHK - 4 * _K)],
                    agg_sh.at[pl.ds(s * _CHK + 4 * _K, _CHK - 4 * _K)])

    @pl.when(s == 0)
    def _():
        pltpu.sync_copy(zstage.at[pl.ds(0, _TAIL)],
                        agg_sh.at[pl.ds(_NS * _CHK, _TAIL)])

    plsc.subcore_barrier()

    coff = jnp.zeros((16,), jnp.int32) + c * _N

    def scale(rows_b, ew_v, q, rows_f):
        def group(g, cc):
            w16 = ew_v[q, pl.ds(g * 16, 16)]
            for e in range(16):
                wb = w16.at[jnp.full((16,), e, jnp.int32)].get(
                    mode="promise_in_bounds")
                r = g * 16 + e
                for d in range(_H // 32):
                    pi = rows_b[r, pl.ds(d * 16, 16)]
                    lo = lax.bitcast_convert_type(pi << 16, jnp.float32)
                    hi = lax.bitcast_convert_type(
                        pi & jnp.int32(-65536), jnp.float32)
                    rows_f[r, pl.ds(d * 32, 16)] = lo * wb
                    rows_f[r, pl.ds(d * 32 + 16, 16)] = hi * wb
            return cc

        lax.fori_loop(0, _K // 16, group, 0, unroll=2)

    def stage(i0, z, sem):
        j0 = i0 * _NBUF
        return [
            pltpu.async_copy(src_hbm.at[s, pl.ds(j0, _NBUF)],
                             src_v2.at[z], sem),
            pltpu.async_copy(dst_hbm.at[s, pl.ds(j0, _NBUF)],
                             dst_v2.at[z], sem),
            pltpu.async_copy(ew_hbm.at[s, pl.ds(j0, _NBUF)],
                             ew_v2.at[z], sem),
        ]

    def process(z):
        src_v = src_v2.at[z]
        dst_v = dst_v2.at[z]
        for r in range(_NBUF):
            for ch in range(_K // 16):
                sl = pl.ds(ch * 16, 16)
                src_v[r, sl] = src_v[r, sl] + coff
        descs = [
            pltpu.async_copy(hp_hbm.at[src_v.at[q]], rows4.at[q], sems[q])
            for q in range(_NBUF)
        ]
        sdescs = [None, None]
        for q in range(_NBUF):
            b = q % 2
            descs[q].wait()
            if sdescs[b] is not None:
                sdescs[b].wait()
            scale(rows4.at[q], ew_v2.at[z], q, rows_f2.at[b])
            sdescs[b] = pltpu.async_copy(
                rows_f2.at[b], agg_sh.at[dst_v.at[q]],
                sems[_NBUF + b], add=True)
        sdescs[0].wait()
        sdescs[1].wait()

    isem_a = sems[_NBUF + 2]
    isem_b = sems[_NBUF + 3]
    for d in stage(0, 0, isem_a):
        d.wait()

    def pair(ii, carry):
        i0 = ii * 2
        db = stage(i0 + 1, 1, isem_b)
        process(0)
        for d in db:
            d.wait()
        da = stage(i0 + 2, 0, isem_a)
        process(1)
        for d in da:
            d.wait()
        return carry

    lax.fori_loop(0, _NBLK // (2 * _NBUF), pair, 0)

    plsc.subcore_barrier()

    @pl.when(c == 0)
    def _():
        pltpu.sync_copy(agg_sh.at[pl.ds(s * _CHK, _CHK)],
                        out0.at[pl.ds(s * _CHK, _CHK)])

        @pl.when(s == 0)
        def _():
            pltpu.sync_copy(agg_sh.at[pl.ds(_NS * _CHK, _TAIL)],
                            out0.at[pl.ds(_NS * _CHK, _TAIL)])

    @pl.when(c == 1)
    def _():
        pltpu.sync_copy(agg_sh.at[pl.ds(s * _CHK, _CHK)],
                        out1.at[pl.ds(s * _CHK, _CHK)])

        @pl.when(s == 0)
        def _():
            pltpu.sync_copy(agg_sh.at[pl.ds(_NS * _CHK, _TAIL)],
                            out1.at[pl.ds(_NS * _CHK, _TAIL)])


def _combine_body(hwh_ref, a0_ref, a1_ref, w_ref, o_ref):
    acc = hwh_ref[...]
    acc = acc + jnp.dot(a0_ref[...], w_ref[0:_H, :],
                        preferred_element_type=jnp.float32)
    acc = acc + jnp.dot(a1_ref[...], w_ref[_H:_D, :],
                        preferred_element_type=jnp.float32)
    o_ref[...] = jnp.maximum(acc, 0.0)


def _combine(hwh, a0, a1, W_tail):
    blk = 1000
    return pl.pallas_call(
        _combine_body,
        grid=(_N // blk,),
        in_specs=[
            pl.BlockSpec((blk, _D), lambda i: (i, 0)),
            pl.BlockSpec((blk, _H), lambda i: (i, 0)),
            pl.BlockSpec((blk, _H), lambda i: (i, 0)),
            pl.BlockSpec((_D, _D), lambda i: (0, 0)),
        ],
        out_specs=pl.BlockSpec((blk, _D), lambda i: (i, 0)),
        out_shape=jax.ShapeDtypeStruct((_N, _D), jnp.float32),
    )(hwh, a0, a1, W_tail)


def _pad_edges(x):
    x2 = x.reshape(_NS, _EPS)
    pad = jnp.zeros((_NS, _EPAD), dtype=x.dtype)
    return jnp.concatenate([x2, pad], axis=1).reshape(_NS, _NBLKP, _K)


def kernel(features, edge_index, edge_weight, W_proj, b_proj, W_agg, b_agg):
    src = _pad_edges(edge_index[0].astype(jnp.int32))
    dst = _pad_edges(edge_index[1].astype(jnp.int32))
    ew = _pad_edges(edge_weight)

    W_tail = jnp.concatenate([
        W_agg[_D:_D + _H][_P64],
        W_agg[_D + _H:2 * _D][_P64],
    ])

    hwh, lo, hi = _project(features, W_proj, b_proj.reshape(1, _D),
                           W_agg[0:_D], b_agg.reshape(1, _D))
    hp = jnp.concatenate([lo, hi], axis=0)
    a0, a1 = _aggregate(src, dst, ew, hp)
    return _combine(hwh, a0, a1, W_tail)


# speedup vs baseline: 1.0171x; 1.0171x over previous
import functools

import jax
import jax.numpy as jnp
import numpy as np
from jax import lax
from jax.experimental import pallas as pl
from jax.experimental.pallas import tpu as pltpu
from jax.experimental.pallas import tpu_sc as plsc

_N = 10000
_D = 128
_H = _D // 2
_E = 320000

_NC = 2
_NS = 16
_EPS = _E // _NS
_K = 128
_NBLK = 160
_NBUF = 8
_NBLKP = _NBLK + _NBUF
_EPAD = _NBLKP * _K - _EPS
_CHK = 624
_TAIL = _N - _NS * _CHK

_P64 = np.r_[np.arange(0, 16), np.arange(32, 48),
             np.arange(16, 32), np.arange(48, 64)]


def _bf16_round_bits(u):
    one = jnp.uint32(1)
    r = u + jnp.uint32(0x7FFF) + ((u >> 16) & one)
    return r & jnp.uint32(0xFFFF0000)


def _project_body(x_ref, wp_ref, bp_ref, wh_ref, ba_ref,
                  o_ref, lo_ref, hi_ref):
    acc = (
        jnp.dot(x_ref[...], wp_ref[...], preferred_element_type=jnp.float32)
        + bp_ref[...]
    )
    o_ref[...] = (
        jnp.dot(acc, wh_ref[...], preferred_element_type=jnp.float32)
        + ba_ref[...]
    )
    bits = _bf16_round_bits(lax.bitcast_convert_type(acc, jnp.uint32))
    lo = (bits[:, 0:32] >> 16) | (bits[:, 32:64])
    hi = (bits[:, 64:96] >> 16) | (bits[:, 96:128])
    lo_ref[...] = lax.bitcast_convert_type(lo, jnp.int32)
    hi_ref[...] = lax.bitcast_convert_type(hi, jnp.int32)


def _project(features, W_proj, b_proj2, W_h, b_agg2):
    blk = 1000
    return pl.pallas_call(
        _project_body,
        grid=(_N // blk,),
        in_specs=[
            pl.BlockSpec((blk, _D), lambda i: (i, 0)),
            pl.BlockSpec((_D, _D), lambda i: (0, 0)),
            pl.BlockSpec((1, _D), lambda i: (0, 0)),
            pl.BlockSpec((_D, _D), lambda i: (0, 0)),
            pl.BlockSpec((1, _D), lambda i: (0, 0)),
        ],
        out_specs=[
            pl.BlockSpec((blk, _D), lambda i: (i, 0)),
            pl.BlockSpec((blk, _H // 2), lambda i: (i, 0)),
            pl.BlockSpec((blk, _H // 2), lambda i: (i, 0)),
        ],
        out_shape=[
            jax.ShapeDtypeStruct((_N, _D), jnp.float32),
            jax.ShapeDtypeStruct((_N, _H // 2), jnp.int32),
            jax.ShapeDtypeStruct((_N, _H // 2), jnp.int32),
        ],
    )(features, W_proj, b_proj2, W_h, b_agg2)


_mesh = plsc.VectorSubcoreMesh(core_axis_name="c", subcore_axis_name="s")


@functools.partial(
    pl.kernel,
    out_type=tuple(
        jax.ShapeDtypeStruct((_N, _H), jnp.float32) for _ in range(2)
    ),
    mesh=_mesh,
    compiler_params=pltpu.CompilerParams(use_tc_tiling_on_sc=False),
    scratch_types=[
        pltpu.VMEM((2, _NBUF, _K), jnp.int32),
        pltpu.VMEM((2, _NBUF, _K), jnp.int32),
        pltpu.VMEM((2, _NBUF, _K), jnp.float32),
        pltpu.VMEM((_NBUF, _K, _H // 2), jnp.int32),
        pltpu.VMEM((2, _K, _H), jnp.float32),
        pltpu.VMEM_SHARED((_N, _H), jnp.float32),
    ] + [pltpu.SemaphoreType.DMA] * (_NBUF + 4),
)
def _aggregate(src_hbm, dst_hbm, ew_hbm, hp_hbm,
               out0, out1,
               src_v2, dst_v2, ew_v2, rows4, rows_f2, agg_sh,
               *sems):
    c = lax.axis_index("c")
    s = lax.axis_index("s")

    zeros = jnp.zeros((16,), jnp.float32)
    zstage = rows_f2.at[0]

    def zrow(r, carry):
        for d in range(_H // 16):
            zstage[r, pl.ds(d * 16, 16)] = zeros
        return carry

    lax.fori_loop(0, _K, zrow, 0)

    for t in range(_CHK // _K):
        pltpu.sync_copy(zstage, agg_sh.at[pl.ds(s * _CHK + t * _K, _K)])
    pltpu.sync_copy(zstage.at[pl.ds(0, _CHK - 4 * _K)],
                    agg_sh.at[pl.ds(s * _CHK + 4 * _K, _CHK - 4 * _K)])

    @pl.when(s == 0)
    def _():
        pltpu.sync_copy(zstage.at[pl.ds(0, _TAIL)],
                        agg_sh.at[pl.ds(_NS * _CHK, _TAIL)])

    plsc.subcore_barrier()

    coff = jnp.zeros((16,), jnp.int32) + c * _N

    def scale(rows_b, ew_v, q, rows_f):
        def group(g, cc):
            w16 = ew_v[q, pl.ds(g * 16, 16)]
            for e in range(16):
                wb = w16.at[jnp.full((16,), e, jnp.int32)].get(
                    mode="promise_in_bounds")
                r = g * 16 + e
                for d in range(_H // 32):
                    pi = rows_b[r, pl.ds(d * 16, 16)]
                    lo = lax.bitcast_convert_type(pi << 16, jnp.float32)
                    hi = lax.bitcast_convert_type(
                        pi & jnp.int32(-65536), jnp.float32)
                    rows_f[r, pl.ds(d * 32, 16)] = lo * wb
                    rows_f[r, pl.ds(d * 32 + 16, 16)] = hi * wb
            return cc

        lax.fori_loop(0, _K // 16, group, 0)

    def stage(i0, z, sem):
        j0 = i0 * _NBUF
        return [
            pltpu.async_copy(src_hbm.at[s, pl.ds(j0, _NBUF)],
                             src_v2.at[z], sem),
            pltpu.async_copy(dst_hbm.at[s, pl.ds(j0, _NBUF)],
                             dst_v2.at[z], sem),
            pltpu.async_copy(ew_hbm.at[s, pl.ds(j0, _NBUF)],
                             ew_v2.at[z], sem),
        ]

    def process(z):
        src_v = src_v2.at[z]
        dst_v = dst_v2.at[z]
        for r in range(_NBUF):
            for ch in range(_K // 16):
                sl = pl.ds(ch * 16, 16)
                src_v[r, sl] = src_v[r, sl] + coff
        descs = [
            pltpu.async_copy(hp_hbm.at[src_v.at[q]], rows4.at[q], sems[q])
            for q in range(_NBUF)
        ]
        sdescs = [None, None]
        for q in range(_NBUF):
            b = q % 2
            descs[q].wait()
            if sdescs[b] is not None:
                sdescs[b].wait()
            scale(rows4.at[q], ew_v2.at[z], q, rows_f2.at[b])
            sdescs[b] = pltpu.async_copy(
                rows_f2.at[b], agg_sh.at[dst_v.at[q]],
                sems[_NBUF + b], add=True)
        sdescs[0].wait()
        sdescs[1].wait()

    isem_a = sems[_NBUF + 2]
    isem_b = sems[_NBUF + 3]
    for d in stage(0, 0, isem_a):
        d.wait()

    def pair(ii, carry):
        i0 = ii * 2
        db = stage(i0 + 1, 1, isem_b)
        process(0)
        for d in db:
            d.wait()
        da = stage(i0 + 2, 0, isem_a)
        process(1)
        for d in da:
            d.wait()
        return carry

    lax.fori_loop(0, _NBLK // (2 * _NBUF), pair, 0)

    plsc.subcore_barrier()

    @pl.when(c == 0)
    def _():
        pltpu.sync_copy(agg_sh.at[pl.ds(s * _CHK, _CHK)],
                        out0.at[pl.ds(s * _CHK, _CHK)])

        @pl.when(s == 0)
        def _():
            pltpu.sync_copy(agg_sh.at[pl.ds(_NS * _CHK, _TAIL)],
                            out0.at[pl.ds(_NS * _CHK, _TAIL)])

    @pl.when(c == 1)
    def _():
        pltpu.sync_copy(agg_sh.at[pl.ds(s * _CHK, _CHK)],
                        out1.at[pl.ds(s * _CHK, _CHK)])

        @pl.when(s == 0)
        def _():
            pltpu.sync_copy(agg_sh.at[pl.ds(_NS * _CHK, _TAIL)],
                            out1.at[pl.ds(_NS * _CHK, _TAIL)])


def _combine_body(hwh_ref, a0_ref, a1_ref, w_ref, o_ref):
    acc = hwh_ref[...]
    acc = acc + jnp.dot(a0_ref[...], w_ref[0:_H, :],
                        preferred_element_type=jnp.float32)
    acc = acc + jnp.dot(a1_ref[...], w_ref[_H:_D, :],
                        preferred_element_type=jnp.float32)
    o_ref[...] = jnp.maximum(acc, 0.0)


def _combine(hwh, a0, a1, W_tail):
    blk = 1000
    return pl.pallas_call(
        _combine_body,
        grid=(_N // blk,),
        in_specs=[
            pl.BlockSpec((blk, _D), lambda i: (i, 0)),
            pl.BlockSpec((blk, _H), lambda i: (i, 0)),
            pl.BlockSpec((blk, _H), lambda i: (i, 0)),
            pl.BlockSpec((_D, _D), lambda i: (0, 0)),
        ],
        out_specs=pl.BlockSpec((blk, _D), lambda i: (i, 0)),
        out_shape=jax.ShapeDtypeStruct((_N, _D), jnp.float32),
    )(hwh, a0, a1, W_tail)


def _pad_edges(x):
    x2 = x.reshape(_NS, _EPS)
    pad = jnp.zeros((_NS, _EPAD), dtype=x.dtype)
    return jnp.concatenate([x2, pad], axis=1).reshape(_NS, _NBLKP, _K)


def kernel(features, edge_index, edge_weight, W_proj, b_proj, W_agg, b_agg):
    src = _pad_edges(edge_index[0].astype(jnp.int32))
    dst = _pad_edges(edge_index[1].astype(jnp.int32))
    ew = _pad_edges(edge_weight)

    W_tail = jnp.concatenate([
        W_agg[_D:_D + _H][_P64],
        W_agg[_D + _H:2 * _D][_P64],
    ])

    hwh, lo, hi = _project(features, W_proj, b_proj.reshape(1, _D),
                           W_agg[0:_D], b_agg.reshape(1, _D))
    hp = jnp.concatenate([lo, hi], axis=0)
    a0, a1 = _aggregate(src, dst, ew, hp)
    return _combine(hwh, a0, a1, W_tail)
